# Initial kernel scaffold; baseline (speedup 1.0000x reference)
import jax
import jax.numpy as jnp
from jax.experimental import pallas as pl


def kernel(x, edge_index, batch, params):
    raise NotImplementedError("write your pallas kernel here")


# speedup vs baseline: 1.0551x; 1.0551x over previous
import functools

import jax
import jax.numpy as jnp
from jax import lax
from jax.experimental import pallas as pl
from jax.experimental.pallas import tpu as pltpu

_N_GRAPHS = 64
_ROW_BLK = 1000


def _mlp_body(h_ref, agg_ref, w1_ref, b1_ref, w2_ref, b2_ref, z_ref, sums_ref):
    i = pl.program_id(0)
    zin = h_ref[...] + agg_ref[...]
    z1 = lax.dot_general(zin, w1_ref[...], (((1,), (0,)), ((), ())),
                         preferred_element_type=jnp.float32) + b1_ref[...]
    z1 = jnp.maximum(z1, 0.0).astype(jnp.bfloat16)
    z2 = lax.dot_general(z1, w2_ref[...], (((1,), (0,)), ((), ())),
                         preferred_element_type=jnp.float32) + b2_ref[...]
    z_ref[...] = z2
    part = jnp.concatenate(
        [jnp.sum(z2, axis=0, keepdims=True),
         jnp.sum(z2 * z2, axis=0, keepdims=True)], axis=0)

    @pl.when(i == 0)
    def _():
        sums_ref[...] = part

    @pl.when(i != 0)
    def _():
        sums_ref[...] += part


def _mlp(h, agg, w1, b1, w2, b2):
    n, din = h.shape
    dh = w1.shape[1]
    grid = n // _ROW_BLK
    return pl.pallas_call(
        _mlp_body,
        grid=(grid,),
        in_specs=[
            pl.BlockSpec((_ROW_BLK, din), lambda i: (i, 0)),
            pl.BlockSpec((_ROW_BLK, din), lambda i: (i, 0)),
            pl.BlockSpec((din, dh), lambda i: (0, 0)),
            pl.BlockSpec((1, dh), lambda i: (0, 0)),
            pl.BlockSpec((dh, dh), lambda i: (0, 0)),
            pl.BlockSpec((1, dh), lambda i: (0, 0)),
        ],
        out_specs=[
            pl.BlockSpec((_ROW_BLK, dh), lambda i: (i, 0)),
            pl.BlockSpec((2, dh), lambda i: (0, 0)),
        ],
        out_shape=[
            jax.ShapeDtypeStruct((n, dh), jnp.float32),
            jax.ShapeDtypeStruct((2, dh), jnp.float32),
        ],
    )(h, agg, w1, b1, w2, b2)


def _bn_body(z_ref, mean_ref, var_ref, gamma_ref, beta_ref, oneh_ref,
             h_ref, pool_ref):
    i = pl.program_id(0)
    zn = ((z_ref[...] - mean_ref[...]) / jnp.sqrt(var_ref[...] + 1e-5)
          * gamma_ref[...] + beta_ref[...])
    h = jnp.maximum(zn, 0.0)
    h_ref[...] = h
    part = lax.dot_general(oneh_ref[...], h, (((0,), (0,)), ((), ())),
                           precision=lax.Precision.HIGHEST,
                           preferred_element_type=jnp.float32)

    @pl.when(i == 0)
    def _():
        pool_ref[...] = part

    @pl.when(i != 0)
    def _():
        pool_ref[...] += part


def _bn_relu_pool(z, mean, var, gamma, beta, oneh):
    n, dh = z.shape
    grid = n // _ROW_BLK
    return pl.pallas_call(
        _bn_body,
        grid=(grid,),
        in_specs=[
            pl.BlockSpec((_ROW_BLK, dh), lambda i: (i, 0)),
            pl.BlockSpec((1, dh), lambda i: (0, 0)),
            pl.BlockSpec((1, dh), lambda i: (0, 0)),
            pl.BlockSpec((1, dh), lambda i: (0, 0)),
            pl.BlockSpec((1, dh), lambda i: (0, 0)),
            pl.BlockSpec((_ROW_BLK, _N_GRAPHS), lambda i: (i, 0)),
        ],
        out_specs=[
            pl.BlockSpec((_ROW_BLK, dh), lambda i: (i, 0)),
            pl.BlockSpec((_N_GRAPHS, dh), lambda i: (0, 0)),
        ],
        out_shape=[
            jax.ShapeDtypeStruct((n, dh), jnp.float32),
            jax.ShapeDtypeStruct((_N_GRAPHS, dh), jnp.float32),
        ],
    )(z, mean, var, gamma, beta, oneh)


def kernel(x, edge_index, batch, params):
    n = x.shape[0]
    src = edge_index[0]
    dst = edge_index[1]
    oneh = (batch[:, None] == jnp.arange(_N_GRAPHS, dtype=jnp.int32)[None, :]
            ).astype(jnp.float32)
    n_layers = sum(1 for k in params if k.startswith('W1_'))
    h = x
    pools = []
    for i in range(n_layers):
        agg = jnp.zeros_like(h).at[dst].add(h[src])
        if i < 2:
            z = h + agg
            z = z @ params['W1_%d' % i] + params['b1_%d' % i]
            z = jax.nn.relu(z)
            z = z @ params['W2_%d' % i] + params['b2_%d' % i]
            mean = jnp.mean(z, axis=0)
            var = jnp.var(z, axis=0)
        else:
            z, sums = _mlp(h, agg,
                           params['W1_%d' % i], params['b1_%d' % i][None, :],
                           params['W2_%d' % i], params['b2_%d' % i][None, :])
            mean = sums[0] * jnp.float32(1.0 / n)
            var = sums[1] * jnp.float32(1.0 / n) - mean * mean
        h, pool = _bn_relu_pool(z, mean[None, :], var[None, :],
                                params['gamma_%d' % i][None, :],
                                params['beta_%d' % i][None, :], oneh)
        pools.append(pool)
    global_rep = jnp.concatenate(pools, axis=1)
    return (global_rep, h)
